# baseline (speedup 1.0000x reference)
import functools

import numpy as np
import jax
import jax.numpy as jnp
from jax import lax
from jax.experimental import pallas as pl
from jax.experimental.pallas import tpu as pltpu
from jax.experimental.pallas import tpu_sc as plsc

_BOX = 100
_B, _C, _H, _W = 64, 3, 224, 224
_NLAB = 1000
_BATCH_PROB = 0.1


def _static_rng():
    rs = np.random.RandomState(42)
    perm = rs.permutation(_B)
    keep = rs.rand(_B) > _BATCH_PROB
    xs = rs.randint(0, _H - _BOX + 1, size=_B)
    ys = rs.randint(0, _W - _BOX + 1, size=_B)
    return perm, keep, xs, ys


_PERM, _KEEP, _XS, _YS = _static_rng()
_KEEP_IDX = np.nonzero(_KEEP)[0]
_K = int(len(_KEEP_IDX))
_LAM = 1.0 - (_BOX * _BOX) / float(_H * _W)

_EYE = np.eye(_B, dtype=np.float32)
_WLAB = (_LAM * _EYE[_KEEP_IDX] + (1.0 - _LAM) * _EYE[_PERM[_KEEP_IDX]])

_NUM_TILES = 32


_L = 16
_BROWS = 112


def _merge_box_rows(pvm, bvm, x, xa, y):
    k_lo = y // _L
    k_hi = (y + _BOX - 1) // _L
    d = x - xa
    col0 = lax.iota(jnp.int32, _L)
    masks = {}
    for k in range(k_lo, k_hi + 1):
        lo = k * _L
        if not (lo >= y and lo + _L <= y + _BOX):
            col = col0 + lo
            masks[k] = (col >= y) & (col < y + _BOX)

    @plsc.parallel_loop(0, _BOX, unroll=1)
    def row(r):
        rb = d + r
        rp = x + r
        for k in range(k_lo, k_hi + 1):
            lo = k * _L
            src = bvm[rb, pl.ds(lo, _L)]
            if k in masks:
                cur = pvm[rp, pl.ds(lo, _L)]
                src = jnp.where(masks[k], src, cur)
            pvm[rp, pl.ds(lo, _L)] = src


def _sc_images_body(images_hbm, out_hbm, pvm, bvm, sem_p, sem_b, sem_o):
    wid = lax.axis_index("s") * 2 + lax.axis_index("c")
    for t in range(_NUM_TILES):
        my = [i for i in range(_K) if i % _NUM_TILES == t]
        if not my:
            continue

        @pl.when(wid == t)
        def _work(my=my):
            for i in my:
                b = int(_KEEP_IDX[i])
                pb = int(_PERM[b])
                x = int(_XS[b])
                y = int(_YS[b])
                xa = min(8 * (x // 8), _H - _BROWS)

                def chan(c, carry, i=i, b=b, pb=pb, x=x, y=y, xa=xa):
                    cb = pltpu.async_copy(
                        images_hbm.at[pb, c, pl.ds(xa, _BROWS)], bvm, sem_b
                    )

                    @pl.when(c > 0)
                    def _drain():
                        pltpu.make_async_copy(
                            pvm, out_hbm.at[i, 0], sem_o
                        ).wait()

                    cp = pltpu.async_copy(images_hbm.at[b, c], pvm, sem_p)
                    cp.wait()
                    cb.wait()
                    _merge_box_rows(pvm, bvm, x, xa, y)
                    pltpu.async_copy(pvm, out_hbm.at[i, c], sem_o)

                    @pl.when(c == _C - 1)
                    def _drain_last():
                        pltpu.make_async_copy(
                            pvm, out_hbm.at[i, 0], sem_o
                        ).wait()

                    return carry

                lax.fori_loop(0, _C, chan, 0)


_sc_images = pl.kernel(
    _sc_images_body,
    out_type=jax.ShapeDtypeStruct((_K, _C, _H, _W), jnp.float32),
    mesh=plsc.VectorSubcoreMesh(core_axis_name="c", subcore_axis_name="s"),
    scratch_types=[
        pltpu.VMEM((_H, _W), jnp.float32),
        pltpu.VMEM((_BROWS, _W), jnp.float32),
        pltpu.SemaphoreType.DMA,
        pltpu.SemaphoreType.DMA,
        pltpu.SemaphoreType.DMA,
    ],
)


def _tc_labels_body(w_ref, l_ref, o_ref):
    o_ref[...] = jnp.dot(
        w_ref[...], l_ref[...], preferred_element_type=jnp.float32
    )


def _tc_labels(labels):
    return pl.pallas_call(
        _tc_labels_body,
        out_shape=jax.ShapeDtypeStruct((_K, _NLAB), jnp.float32),
    )(jnp.asarray(_WLAB), labels)


@jax.jit
def kernel(images, labels):
    mixed = _sc_images(images)
    mixed_labels = _tc_labels(labels)
    return mixed, mixed_labels
